# baseline (speedup 1.0000x reference)
import functools

import jax
import jax.numpy as jnp
from jax.experimental import pallas as pl
from jax.experimental.pallas import tpu as pltpu

CHS = [96, 192, 384]
HWS = [64, 32, 16]
B = 4
NC = 80
EMB = 128


def _silu(x):
    return x * jax.nn.sigmoid(x)


def _shift_h(v, dx, W):
    if dx == 0:
        return v
    z = jnp.zeros_like(v[:, :1, :])
    if dx == 1:
        return jnp.concatenate([v[:, 1:, :], z], axis=1)
    return jnp.concatenate([z, v[:, : W - 1, :]], axis=1)


def _conv3x3(xpad, wt, H, W):
    acc = None
    for ky in range(3):
        v = xpad[ky : ky + H]
        for kx in range(3):
            s = _shift_h(v, kx - 1, W)
            t = jnp.dot(
                s.reshape(H * W, -1), wt[ky, kx],
                preferred_element_type=jnp.float32,
            )
            acc = t if acc is None else acc + t
    return acc


def _head_kernel(
    x_ref,
    cw0_ref, cb0_ref, cw1_ref, cb1_ref,
    rw0_ref, rb0_ref, rw1_ref, rb1_ref,
    cpw_ref, cpb_ref, rpw_ref, rpb_ref, opw_ref, opb_ref,
    ew_ref, eb_ref, epw_ref, epb_ref,
    cls_ref, reg_ref, obj_ref, emb_ref,
    xpad_ref,
    *, H, W,
):
    C = x_ref.shape[-1]
    xpad_ref[0] = jnp.zeros((W, C), jnp.float32)
    xpad_ref[H + 1] = jnp.zeros((W, C), jnp.float32)
    xpad_ref[1 : H + 1] = x_ref[0]

    def conv_block(wt_ref, b_ref):
        y = _conv3x3(xpad_ref[...], wt_ref[...], H, W) + b_ref[...]
        y = _silu(y)
        xpad_ref[1 : H + 1] = y.reshape(H, W, C)
        return y

    c1 = conv_block(cw0_ref, cb0_ref)
    del c1
    c2 = conv_block(cw1_ref, cb1_ref)
    cls_ref[0] = (
        jnp.dot(c2, cpw_ref[...], preferred_element_type=jnp.float32)
        + cpb_ref[...]
    )

    xpad_ref[1 : H + 1] = x_ref[0]
    r1 = conv_block(rw0_ref, rb0_ref)
    del r1
    r2 = conv_block(rw1_ref, rb1_ref)
    reg_ref[0] = (
        jnp.dot(r2, rpw_ref[...], preferred_element_type=jnp.float32)
        + rpb_ref[...]
    )
    obj_ref[0] = (
        jnp.dot(r2, opw_ref[...], preferred_element_type=jnp.float32)
        + opb_ref[...]
    )

    xpad_ref[1 : H + 1] = x_ref[0]
    e1 = conv_block(ew_ref, eb_ref)
    e = (
        jnp.dot(e1, epw_ref[...], preferred_element_type=jnp.float32)
        + epb_ref[...]
    )
    n = jnp.sqrt(jnp.sum(e * e, axis=1, keepdims=True))
    emb_ref[0] = e / jnp.maximum(n, 1e-12)


def _scale_head(feat, cw0, cb0, cw1, cb1, rw0, rb0, rw1, rb1,
                cpw, cpb, rpw, rpb, opw, opb, ew, eb, epw, epb):
    Bn, C, H, W = feat.shape
    xt = feat.transpose(0, 2, 3, 1)

    def wt(w):
        return w.transpose(2, 3, 1, 0)

    def pwt(w):
        return w[:, :, 0, 0].T

    def row(b):
        return b.reshape(1, -1)

    args = (
        xt,
        wt(cw0), row(cb0), wt(cw1), row(cb1),
        wt(rw0), row(rb0), wt(rw1), row(rb1),
        pwt(cpw), row(cpb), pwt(rpw), row(rpb), pwt(opw), row(opb),
        wt(ew), row(eb), pwt(epw), row(epb),
    )

    const = lambda shape: pl.BlockSpec(shape, lambda b: (0,) * len(shape))
    in_specs = [pl.BlockSpec((1, H, W, C), lambda b: (b, 0, 0, 0))]
    for a in args[1:]:
        in_specs.append(const(a.shape))

    out_shapes = [
        jax.ShapeDtypeStruct((Bn, H * W, NC), jnp.float32),
        jax.ShapeDtypeStruct((Bn, H * W, 4), jnp.float32),
        jax.ShapeDtypeStruct((Bn, H * W, 1), jnp.float32),
        jax.ShapeDtypeStruct((Bn, H * W, EMB), jnp.float32),
    ]
    out_specs = [
        pl.BlockSpec((1, H * W, s.shape[-1]), lambda b: (b, 0, 0))
        for s in out_shapes
    ]

    cls, reg, obj, emb = pl.pallas_call(
        functools.partial(_head_kernel, H=H, W=W),
        grid=(Bn,),
        in_specs=in_specs,
        out_specs=out_specs,
        out_shape=out_shapes,
        scratch_shapes=[pltpu.VMEM((H + 2, W, C), jnp.float32)],
    )(*args)

    def to_nchw(y):
        return y.reshape(Bn, H, W, -1).transpose(0, 3, 1, 2)

    return to_nchw(cls), to_nchw(reg), to_nchw(obj), to_nchw(emb)


def kernel(feat0, feat1, feat2,
           cls_w_0_0, cls_b_0_0, cls_w_0_1, cls_b_0_1,
           reg_w_0_0, reg_b_0_0, reg_w_0_1, reg_b_0_1,
           cls_pw_0, cls_pb_0, reg_pw_0, reg_pb_0, obj_pw_0, obj_pb_0,
           emb_w_0, emb_b_0, emb_pw_0, emb_pb_0,
           cls_w_1_0, cls_b_1_0, cls_w_1_1, cls_b_1_1,
           reg_w_1_0, reg_b_1_0, reg_w_1_1, reg_b_1_1,
           cls_pw_1, cls_pb_1, reg_pw_1, reg_pb_1, obj_pw_1, obj_pb_1,
           emb_w_1, emb_b_1, emb_pw_1, emb_pb_1,
           cls_w_2_0, cls_b_2_0, cls_w_2_1, cls_b_2_1,
           reg_w_2_0, reg_b_2_0, reg_w_2_1, reg_b_2_1,
           cls_pw_2, cls_pb_2, reg_pw_2, reg_pb_2, obj_pw_2, obj_pb_2,
           emb_w_2, emb_b_2, emb_pw_2, emb_pb_2):
    feats = [feat0, feat1, feat2]
    p = dict(locals())
    cls_outs, reg_outs, obj_outs, emb_outs = [], [], [], []
    for i, feat in enumerate(feats):
        c, r, o, e = _scale_head(
            feat,
            p[f'cls_w_{i}_0'], p[f'cls_b_{i}_0'],
            p[f'cls_w_{i}_1'], p[f'cls_b_{i}_1'],
            p[f'reg_w_{i}_0'], p[f'reg_b_{i}_0'],
            p[f'reg_w_{i}_1'], p[f'reg_b_{i}_1'],
            p[f'cls_pw_{i}'], p[f'cls_pb_{i}'],
            p[f'reg_pw_{i}'], p[f'reg_pb_{i}'],
            p[f'obj_pw_{i}'], p[f'obj_pb_{i}'],
            p[f'emb_w_{i}'], p[f'emb_b_{i}'],
            p[f'emb_pw_{i}'], p[f'emb_pb_{i}'],
        )
        cls_outs.append(c)
        reg_outs.append(r)
        obj_outs.append(o)
        emb_outs.append(e)
    return tuple(cls_outs + reg_outs + obj_outs + emb_outs)


# speedup vs baseline: 1.0999x; 1.0999x over previous
import functools

import jax
import jax.numpy as jnp
from jax.experimental import pallas as pl
from jax.experimental.pallas import tpu as pltpu

CHS = [96, 192, 384]
HWS = [64, 32, 16]
B = 4
NC = 80
EMB = 128


def _silu(x):
    return x * jax.nn.sigmoid(x)


def _shift_h(v, dx, W):
    if dx == 0:
        return v
    z = jnp.zeros_like(v[:, :1, :])
    if dx == 1:
        return jnp.concatenate([v[:, 1:, :], z], axis=1)
    return jnp.concatenate([z, v[:, : W - 1, :]], axis=1)


def _conv3x3(xpad, wt, H, W):
    acc = None
    for ky in range(3):
        v = xpad[ky : ky + H]
        for kx in range(3):
            s = _shift_h(v, kx - 1, W)
            t = jnp.dot(
                s.reshape(H * W, -1), wt[ky, kx],
                preferred_element_type=jnp.float32,
            )
            acc = t if acc is None else acc + t
    return acc


def _head_kernel(
    x_ref,
    cw0_ref, cb0_ref, cw1_ref, cb1_ref,
    rw0_ref, rb0_ref, rw1_ref, rb1_ref,
    cpw_ref, cpb_ref, rpw_ref, rpb_ref, opw_ref, opb_ref,
    ew_ref, eb_ref, epw_ref, epb_ref,
    cls_ref, reg_ref, obj_ref, emb_ref,
    xpad_ref,
    *, H, W,
):
    C = x_ref.shape[-1]
    xpad_ref[0] = jnp.zeros((W, C), jnp.bfloat16)
    xpad_ref[H + 1] = jnp.zeros((W, C), jnp.bfloat16)
    xpad_ref[1 : H + 1] = x_ref[0].astype(jnp.bfloat16)

    def conv_block(wt_ref, b_ref):
        y = _conv3x3(xpad_ref[...], wt_ref[...], H, W) + b_ref[...]
        y = _silu(y)
        yb = y.astype(jnp.bfloat16)
        xpad_ref[1 : H + 1] = yb.reshape(H, W, C)
        return yb

    c1 = conv_block(cw0_ref, cb0_ref)
    del c1
    c2 = conv_block(cw1_ref, cb1_ref)
    cls_ref[0] = (
        jnp.dot(c2, cpw_ref[...], preferred_element_type=jnp.float32)
        + cpb_ref[...]
    )

    xpad_ref[1 : H + 1] = x_ref[0].astype(jnp.bfloat16)
    r1 = conv_block(rw0_ref, rb0_ref)
    del r1
    r2 = conv_block(rw1_ref, rb1_ref)
    reg_ref[0] = (
        jnp.dot(r2, rpw_ref[...], preferred_element_type=jnp.float32)
        + rpb_ref[...]
    )
    obj_ref[0] = (
        jnp.dot(r2, opw_ref[...], preferred_element_type=jnp.float32)
        + opb_ref[...]
    )

    xpad_ref[1 : H + 1] = x_ref[0].astype(jnp.bfloat16)
    e1 = conv_block(ew_ref, eb_ref)
    e = (
        jnp.dot(e1, epw_ref[...], preferred_element_type=jnp.float32)
        + epb_ref[...]
    )
    n = jnp.sqrt(jnp.sum(e * e, axis=1, keepdims=True))
    emb_ref[0] = e / jnp.maximum(n, 1e-12)


def _scale_head(feat, cw0, cb0, cw1, cb1, rw0, rb0, rw1, rb1,
                cpw, cpb, rpw, rpb, opw, opb, ew, eb, epw, epb):
    Bn, C, H, W = feat.shape
    xt = feat.transpose(0, 2, 3, 1)

    def wt(w):
        return w.transpose(2, 3, 1, 0)

    def pwt(w):
        return w[:, :, 0, 0].T

    def row(b):
        return b.reshape(1, -1)

    bf = lambda a: a.astype(jnp.bfloat16)
    args = (
        xt,
        bf(wt(cw0)), row(cb0), bf(wt(cw1)), row(cb1),
        bf(wt(rw0)), row(rb0), bf(wt(rw1)), row(rb1),
        bf(pwt(cpw)), row(cpb), bf(pwt(rpw)), row(rpb),
        bf(pwt(opw)), row(opb),
        bf(wt(ew)), row(eb), bf(pwt(epw)), row(epb),
    )

    const = lambda shape: pl.BlockSpec(shape, lambda b: (0,) * len(shape))
    in_specs = [pl.BlockSpec((1, H, W, C), lambda b: (b, 0, 0, 0))]
    for a in args[1:]:
        in_specs.append(const(a.shape))

    out_shapes = [
        jax.ShapeDtypeStruct((Bn, H * W, NC), jnp.float32),
        jax.ShapeDtypeStruct((Bn, H * W, 4), jnp.float32),
        jax.ShapeDtypeStruct((Bn, H * W, 1), jnp.float32),
        jax.ShapeDtypeStruct((Bn, H * W, EMB), jnp.float32),
    ]
    out_specs = [
        pl.BlockSpec((1, H * W, s.shape[-1]), lambda b: (b, 0, 0))
        for s in out_shapes
    ]

    cls, reg, obj, emb = pl.pallas_call(
        functools.partial(_head_kernel, H=H, W=W),
        grid=(Bn,),
        in_specs=in_specs,
        out_specs=out_specs,
        out_shape=out_shapes,
        scratch_shapes=[pltpu.VMEM((H + 2, W, C), jnp.bfloat16)],
    )(*args)

    def to_nchw(y):
        return y.reshape(Bn, H, W, -1).transpose(0, 3, 1, 2)

    return to_nchw(cls), to_nchw(reg), to_nchw(obj), to_nchw(emb)


def kernel(feat0, feat1, feat2,
           cls_w_0_0, cls_b_0_0, cls_w_0_1, cls_b_0_1,
           reg_w_0_0, reg_b_0_0, reg_w_0_1, reg_b_0_1,
           cls_pw_0, cls_pb_0, reg_pw_0, reg_pb_0, obj_pw_0, obj_pb_0,
           emb_w_0, emb_b_0, emb_pw_0, emb_pb_0,
           cls_w_1_0, cls_b_1_0, cls_w_1_1, cls_b_1_1,
           reg_w_1_0, reg_b_1_0, reg_w_1_1, reg_b_1_1,
           cls_pw_1, cls_pb_1, reg_pw_1, reg_pb_1, obj_pw_1, obj_pb_1,
           emb_w_1, emb_b_1, emb_pw_1, emb_pb_1,
           cls_w_2_0, cls_b_2_0, cls_w_2_1, cls_b_2_1,
           reg_w_2_0, reg_b_2_0, reg_w_2_1, reg_b_2_1,
           cls_pw_2, cls_pb_2, reg_pw_2, reg_pb_2, obj_pw_2, obj_pb_2,
           emb_w_2, emb_b_2, emb_pw_2, emb_pb_2):
    feats = [feat0, feat1, feat2]
    p = dict(locals())
    cls_outs, reg_outs, obj_outs, emb_outs = [], [], [], []
    for i, feat in enumerate(feats):
        c, r, o, e = _scale_head(
            feat,
            p[f'cls_w_{i}_0'], p[f'cls_b_{i}_0'],
            p[f'cls_w_{i}_1'], p[f'cls_b_{i}_1'],
            p[f'reg_w_{i}_0'], p[f'reg_b_{i}_0'],
            p[f'reg_w_{i}_1'], p[f'reg_b_{i}_1'],
            p[f'cls_pw_{i}'], p[f'cls_pb_{i}'],
            p[f'reg_pw_{i}'], p[f'reg_pb_{i}'],
            p[f'obj_pw_{i}'], p[f'obj_pb_{i}'],
            p[f'emb_w_{i}'], p[f'emb_b_{i}'],
            p[f'emb_pw_{i}'], p[f'emb_pb_{i}'],
        )
        cls_outs.append(c)
        reg_outs.append(r)
        obj_outs.append(o)
        emb_outs.append(e)
    return tuple(cls_outs + reg_outs + obj_outs + emb_outs)


# speedup vs baseline: 1.1905x; 1.0823x over previous
import functools

import jax
import jax.numpy as jnp
from jax.experimental import pallas as pl
from jax.experimental.pallas import tpu as pltpu

NC = 80
EMB = 128


def _silu(x):
    return x * jax.nn.sigmoid(x)


def _shift_h(v, dx, W):
    if dx == 0:
        return v
    z = jnp.zeros_like(v[:, :1, :])
    if dx == 1:
        return jnp.concatenate([v[:, 1:, :], z], axis=1)
    return jnp.concatenate([z, v[:, : W - 1, :]], axis=1)


def _head_kernel(
    x_ref,
    w1_ref, b1_ref,
    wc2_ref, bc2_ref, wr2_ref, br2_ref,
    cpw_ref, cpb_ref, rpow_ref, rpob_ref, epw_ref, epb_ref,
    cls_ref, reg_ref, obj_ref, emb_ref,
    xp_ref, xc_ref, y1_ref,
    *, H, W,
):
    C = x_ref.shape[-1]
    HW = H * W

    def build_xc():
        for ky in range(3):
            v = xp_ref[ky : ky + H]
            for kx in range(3):
                s = _shift_h(v, kx - 1, W)
                idx = 3 * ky + kx
                xc_ref[:, idx * C : (idx + 1) * C] = s.reshape(HW, C)

    def mm(a, b_ref):
        return jnp.dot(a, b_ref[...], preferred_element_type=jnp.float32)

    xp_ref[0] = jnp.zeros((W, C), jnp.bfloat16)
    xp_ref[H + 1] = jnp.zeros((W, C), jnp.bfloat16)
    xp_ref[1 : H + 1] = x_ref[0].astype(jnp.bfloat16)
    build_xc()

    y1_ref[...] = _silu(mm(xc_ref[...], w1_ref) + b1_ref[...]).astype(
        jnp.bfloat16
    )

    e = mm(y1_ref[:, 2 * C : 3 * C], epw_ref) + epb_ref[...]
    n = jnp.sqrt(jnp.sum(e * e, axis=1, keepdims=True))
    emb_ref[0] = e / jnp.maximum(n, 1e-12)

    xp_ref[1 : H + 1] = y1_ref[:, 0:C].reshape(H, W, C)
    build_xc()
    c2 = _silu(mm(xc_ref[...], wc2_ref) + bc2_ref[...]).astype(jnp.bfloat16)
    cls_ref[0] = mm(c2, cpw_ref) + cpb_ref[...]

    xp_ref[1 : H + 1] = y1_ref[:, C : 2 * C].reshape(H, W, C)
    build_xc()
    r2 = _silu(mm(xc_ref[...], wr2_ref) + br2_ref[...]).astype(jnp.bfloat16)
    t = mm(r2, rpow_ref) + rpob_ref[...]
    reg_ref[0] = t[:, 0:4]
    obj_ref[0] = t[:, 4:5]


def _scale_head(feat, cw0, cb0, cw1, cb1, rw0, rb0, rw1, rb1,
                cpw, cpb, rpw, rpb, opw, opb, ew, eb, epw, epb):
    Bn, C, H, W = feat.shape
    xt = feat.transpose(0, 2, 3, 1)

    def wt9(w):
        return w.transpose(2, 3, 1, 0).reshape(9 * C, -1).astype(jnp.bfloat16)

    def pwt(w):
        return w[:, :, 0, 0].T.astype(jnp.bfloat16)

    def row(*bs):
        return jnp.concatenate(bs).reshape(1, -1)

    w1 = jnp.concatenate([wt9(cw0), wt9(rw0), wt9(ew)], axis=1)
    args = (
        xt,
        w1, row(cb0, rb0, eb),
        wt9(cw1), row(cb1), wt9(rw1), row(rb1),
        pwt(cpw), row(cpb),
        jnp.concatenate([pwt(rpw), pwt(opw)], axis=1), row(rpb, opb),
        pwt(epw), row(epb),
    )

    const = lambda shape: pl.BlockSpec(shape, lambda b: (0,) * len(shape))
    in_specs = [pl.BlockSpec((1, H, W, C), lambda b: (b, 0, 0, 0))]
    for a in args[1:]:
        in_specs.append(const(a.shape))

    out_shapes = [
        jax.ShapeDtypeStruct((Bn, H * W, NC), jnp.float32),
        jax.ShapeDtypeStruct((Bn, H * W, 4), jnp.float32),
        jax.ShapeDtypeStruct((Bn, H * W, 1), jnp.float32),
        jax.ShapeDtypeStruct((Bn, H * W, EMB), jnp.float32),
    ]
    out_specs = [
        pl.BlockSpec((1, H * W, s.shape[-1]), lambda b: (b, 0, 0))
        for s in out_shapes
    ]

    cls, reg, obj, emb = pl.pallas_call(
        functools.partial(_head_kernel, H=H, W=W),
        grid=(Bn,),
        in_specs=in_specs,
        out_specs=out_specs,
        out_shape=out_shapes,
        scratch_shapes=[
            pltpu.VMEM((H + 2, W, C), jnp.bfloat16),
            pltpu.VMEM((H * W, 9 * C), jnp.bfloat16),
            pltpu.VMEM((H * W, 3 * C), jnp.bfloat16),
        ],
    )(*args)

    def to_nchw(y):
        return y.reshape(Bn, H, W, -1).transpose(0, 3, 1, 2)

    return to_nchw(cls), to_nchw(reg), to_nchw(obj), to_nchw(emb)


def kernel(feat0, feat1, feat2,
           cls_w_0_0, cls_b_0_0, cls_w_0_1, cls_b_0_1,
           reg_w_0_0, reg_b_0_0, reg_w_0_1, reg_b_0_1,
           cls_pw_0, cls_pb_0, reg_pw_0, reg_pb_0, obj_pw_0, obj_pb_0,
           emb_w_0, emb_b_0, emb_pw_0, emb_pb_0,
           cls_w_1_0, cls_b_1_0, cls_w_1_1, cls_b_1_1,
           reg_w_1_0, reg_b_1_0, reg_w_1_1, reg_b_1_1,
           cls_pw_1, cls_pb_1, reg_pw_1, reg_pb_1, obj_pw_1, obj_pb_1,
           emb_w_1, emb_b_1, emb_pw_1, emb_pb_1,
           cls_w_2_0, cls_b_2_0, cls_w_2_1, cls_b_2_1,
           reg_w_2_0, reg_b_2_0, reg_w_2_1, reg_b_2_1,
           cls_pw_2, cls_pb_2, reg_pw_2, reg_pb_2, obj_pw_2, obj_pb_2,
           emb_w_2, emb_b_2, emb_pw_2, emb_pb_2):
    feats = [feat0, feat1, feat2]
    p = dict(locals())
    cls_outs, reg_outs, obj_outs, emb_outs = [], [], [], []
    for i, feat in enumerate(feats):
        c, r, o, e = _scale_head(
            feat,
            p[f'cls_w_{i}_0'], p[f'cls_b_{i}_0'],
            p[f'cls_w_{i}_1'], p[f'cls_b_{i}_1'],
            p[f'reg_w_{i}_0'], p[f'reg_b_{i}_0'],
            p[f'reg_w_{i}_1'], p[f'reg_b_{i}_1'],
            p[f'cls_pw_{i}'], p[f'cls_pb_{i}'],
            p[f'reg_pw_{i}'], p[f'reg_pb_{i}'],
            p[f'obj_pw_{i}'], p[f'obj_pb_{i}'],
            p[f'emb_w_{i}'], p[f'emb_b_{i}'],
            p[f'emb_pw_{i}'], p[f'emb_pb_{i}'],
        )
        cls_outs.append(c)
        reg_outs.append(r)
        obj_outs.append(o)
        emb_outs.append(e)
    return tuple(cls_outs + reg_outs + obj_outs + emb_outs)
